# baseline (speedup 1.0000x reference)
import functools

import jax
import jax.numpy as jnp
from jax import lax
from jax.experimental import pallas as pl
from jax.experimental.pallas import tpu as pltpu
from jax.experimental.pallas import tpu_sc as plsc

_B = 8
_N = 8192
_M = 4096
_L = 16
_CK = 8
_CB = _CK * _L

_NSC = 2048
_SH = 4
_RS = _NSC // _SH

_RB = 1024
_NTC = _N - _NSC
_NRB = _NTC // _RB

_mesh = plsc.VectorSubcoreMesh(core_axis_name="c", subcore_axis_name="s")

_GDN = lax.GatherDimensionNumbers(
    offset_dims=(), collapsed_slice_dims=(0,), start_index_map=(0,))


def _lane_perm(x, idx):
    return lax.gather(x, idx[:, None], _GDN, (1,),
                      indices_are_sorted=False, unique_indices=False,
                      mode=lax.GatherScatterMode.PROMISE_IN_BOUNDS)


@functools.partial(
    pl.kernel,
    mesh=_mesh,
    out_type=[
        jax.ShapeDtypeStruct((_B, _NSC), jnp.float32),
        jax.ShapeDtypeStruct((_B, _SH, _M), jnp.float32),
    ],
    scratch_types=[
        pltpu.VMEM((1, _RS), jnp.float32),
        pltpu.VMEM((1, _RS), jnp.float32),
        pltpu.VMEM((1, _RS), jnp.float32),
        pltpu.VMEM((1, _M), jnp.float32),
        pltpu.VMEM((1, _M), jnp.float32),
        pltpu.VMEM((1, _M), jnp.float32),
        pltpu.VMEM((_RS,), jnp.float32),
        pltpu.VMEM((_M,), jnp.float32),
    ],
)
def _sc_min(at_h, bt_h, row_o, col_o,
            axv, ayv, azv, bxv, byv, bzv, rminv, cminv):
    wid = lax.axis_index("s") * 2 + lax.axis_index("c")
    box = wid // _SH
    sh = wid % _SH
    r0 = sh * _RS

    pltpu.sync_copy(at_h.at[box, pl.ds(0, 1), pl.ds(r0, _RS)], axv)
    pltpu.sync_copy(at_h.at[box, pl.ds(1, 1), pl.ds(r0, _RS)], ayv)
    pltpu.sync_copy(at_h.at[box, pl.ds(2, 1), pl.ds(r0, _RS)], azv)
    pltpu.sync_copy(bt_h.at[box, pl.ds(0, 1)], bxv)
    pltpu.sync_copy(bt_h.at[box, pl.ds(1, 1)], byv)
    pltpu.sync_copy(bt_h.at[box, pl.ds(2, 1)], bzv)

    inf16 = jnp.full((_L,), jnp.inf, jnp.float32)
    lanes = lax.iota(jnp.int32, _L)
    xors = [lanes ^ st for st in (1, 2, 4, 8)]

    def initg(g, c):
        rminv[pl.ds(g * _L, _L)] = inf16
        return c

    lax.fori_loop(0, _RS // _L, initg, 0)

    def cb_body(cb, c):
        c0 = cb * _CB
        bxs = [bxv[0, pl.ds(c0 + j * _L, _L)] for j in range(_CK)]
        bys = [byv[0, pl.ds(c0 + j * _L, _L)] for j in range(_CK)]
        bzs = [bzv[0, pl.ds(c0 + j * _L, _L)] for j in range(_CK)]

        def g_body(g, colaccs):
            ax16 = axv[0, pl.ds(g * _L, _L)]
            ay16 = ayv[0, pl.ds(g * _L, _L)]
            az16 = azv[0, pl.ds(g * _L, _L)]
            rvec = inf16
            accs = list(colaccs)
            for r in range(_L):
                axr = ax16[r]
                ayr = ay16[r]
                azr = az16[r]
                rr = None
                for j in range(_CK):
                    dx = bxs[j] - axr
                    dy = bys[j] - ayr
                    dz = bzs[j] - azr
                    d2 = dx * dx + dy * dy + dz * dz
                    accs[j] = jnp.minimum(accs[j], d2)
                    rr = d2 if rr is None else jnp.minimum(rr, d2)
                for idxv in xors:
                    rr = jnp.minimum(rr, _lane_perm(rr, idxv))
                rvec = jnp.where(lanes == r, rr, rvec)
            rminv[pl.ds(g * _L, _L)] = jnp.minimum(rminv[pl.ds(g * _L, _L)],
                                                   rvec)
            return tuple(accs)

        colaccs = lax.fori_loop(0, _RS // _L, g_body, (inf16,) * _CK)
        for j in range(_CK):
            cminv[pl.ds(c0 + j * _L, _L)] = colaccs[j]
        return c

    lax.fori_loop(0, _M // _CB, cb_body, 0)

    pltpu.sync_copy(rminv, row_o.at[box, pl.ds(r0, _RS)])
    pltpu.sync_copy(cminv, col_o.at[box, sh])


def _tp_a_body(a_ref, at_ref):
    at_ref[0] = a_ref[0].T


_tp_a = pl.pallas_call(
    _tp_a_body,
    grid=(_B,),
    in_specs=[pl.BlockSpec((1, _N, 3), lambda b: (b, 0, 0))],
    out_specs=pl.BlockSpec((1, 3, _N), lambda b: (b, 0, 0)),
    out_shape=jax.ShapeDtypeStruct((_B, 3, _N), jnp.float32),
)


def _tp_b_body(b_ref, bt_ref):
    bt_ref[0] = b_ref[0].T


_tp_b = pl.pallas_call(
    _tp_b_body,
    grid=(_B,),
    in_specs=[pl.BlockSpec((1, _M, 3), lambda b: (b, 0, 0))],
    out_specs=pl.BlockSpec((1, 3, _M), lambda b: (b, 0, 0)),
    out_shape=jax.ShapeDtypeStruct((_B, 3, _M), jnp.float32),
)


def _tc_min_body(a_ref, bt_ref, rowmin_ref, colminp_ref):
    axb = a_ref[0, :, 0:1]
    ayb = a_ref[0, :, 1:2]
    azb = a_ref[0, :, 2:3]
    bxb = bt_ref[0, 0:1, :]
    byb = bt_ref[0, 1:2, :]
    bzb = bt_ref[0, 2:3, :]
    d2 = (jnp.square(axb - bxb) + jnp.square(ayb - byb)
          + jnp.square(azb - bzb))
    rowmin_ref[0, 0, 0] = jnp.min(d2, axis=1)
    colminp_ref[0, 0, 0] = jnp.min(d2, axis=0)


_tc_min = pl.pallas_call(
    _tc_min_body,
    grid=(_B, _NRB),
    in_specs=[
        pl.BlockSpec((1, _RB, 3), lambda b, r: (b, (_NSC // _RB) + r, 0)),
        pl.BlockSpec((1, 3, _M), lambda b, r: (b, 0, 0)),
    ],
    out_specs=[
        pl.BlockSpec((1, 1, 1, _RB), lambda b, r: (b, r, 0, 0)),
        pl.BlockSpec((1, 1, 1, _M), lambda b, r: (b, r, 0, 0)),
    ],
    out_shape=[
        jax.ShapeDtypeStruct((_B, _NRB, 1, _RB), jnp.float32),
        jax.ShapeDtypeStruct((_B, _NRB, 1, _M), jnp.float32),
    ],
)


def _finish_body(rs_ref, rt_ref, cs_ref, ct_ref, out_ref):
    rt2 = rt_ref[...].reshape(_B, _NTC)
    rm2 = jnp.concatenate([rs_ref[...], rt2], axis=1)
    cm2 = jnp.minimum(jnp.min(cs_ref[...], axis=1),
                      jnp.min(ct_ref[...].reshape(_B, _NRB, _M), axis=1))
    rm = jnp.sqrt(jnp.maximum(rm2, 1e-12))
    cm = jnp.sqrt(jnp.maximum(cm2, 1e-12))
    lb = jnp.mean(cm, axis=1)
    lf = jnp.mean(rm, axis=1)
    lm = jnp.max(rm, axis=1)
    out_ref[...] = jnp.mean(5.0 * lb + lf + lm).reshape(1, 1)


_finish = pl.pallas_call(
    _finish_body,
    out_shape=jax.ShapeDtypeStruct((1, 1), jnp.float32),
)


@jax.jit
def kernel(sampled_lidar_list, surface_points):
    a = sampled_lidar_list
    b = surface_points
    bt = _tp_b(b)
    at = _tp_a(a)
    rowmin_sc, colmin_sc = _sc_min(at, bt)
    rowmin_tc, colmin_tc = _tc_min(a, bt)
    return _finish(rowmin_sc, rowmin_tc, colmin_sc, colmin_tc)[0, 0]


# speedup vs baseline: 1.0118x; 1.0118x over previous
import functools

import jax
import jax.numpy as jnp
from jax import lax
from jax.experimental import pallas as pl
from jax.experimental.pallas import tpu as pltpu
from jax.experimental.pallas import tpu_sc as plsc

_B = 8
_N = 8192
_M = 4096
_L = 16
_CK = 8
_CB = _CK * _L

_NSC = 2560
_SH = 4
_RS = _NSC // _SH

_RB = 512
_NTC = _N - _NSC
_NRB = _NTC // _RB

_mesh = plsc.VectorSubcoreMesh(core_axis_name="c", subcore_axis_name="s")

_GDN = lax.GatherDimensionNumbers(
    offset_dims=(), collapsed_slice_dims=(0,), start_index_map=(0,))


def _lane_perm(x, idx):
    return lax.gather(x, idx[:, None], _GDN, (1,),
                      indices_are_sorted=False, unique_indices=False,
                      mode=lax.GatherScatterMode.PROMISE_IN_BOUNDS)


@functools.partial(
    pl.kernel,
    mesh=_mesh,
    out_type=[
        jax.ShapeDtypeStruct((_B, _NSC), jnp.float32),
        jax.ShapeDtypeStruct((_B, _SH, _M), jnp.float32),
    ],
    scratch_types=[
        pltpu.VMEM((1, _RS), jnp.float32),
        pltpu.VMEM((1, _RS), jnp.float32),
        pltpu.VMEM((1, _RS), jnp.float32),
        pltpu.VMEM((1, _M), jnp.float32),
        pltpu.VMEM((1, _M), jnp.float32),
        pltpu.VMEM((1, _M), jnp.float32),
        pltpu.VMEM((_RS,), jnp.float32),
        pltpu.VMEM((_M,), jnp.float32),
    ],
)
def _sc_min(at_h, bt_h, row_o, col_o,
            axv, ayv, azv, bxv, byv, bzv, rminv, cminv):
    wid = lax.axis_index("s") * 2 + lax.axis_index("c")
    box = wid // _SH
    sh = wid % _SH
    r0 = sh * _RS

    pltpu.sync_copy(at_h.at[box, pl.ds(0, 1), pl.ds(r0, _RS)], axv)
    pltpu.sync_copy(at_h.at[box, pl.ds(1, 1), pl.ds(r0, _RS)], ayv)
    pltpu.sync_copy(at_h.at[box, pl.ds(2, 1), pl.ds(r0, _RS)], azv)
    pltpu.sync_copy(bt_h.at[box, pl.ds(0, 1)], bxv)
    pltpu.sync_copy(bt_h.at[box, pl.ds(1, 1)], byv)
    pltpu.sync_copy(bt_h.at[box, pl.ds(2, 1)], bzv)

    inf16 = jnp.full((_L,), jnp.inf, jnp.float32)
    lanes = lax.iota(jnp.int32, _L)
    xors = [lanes ^ st for st in (1, 2, 4, 8)]

    def initg(g, c):
        rminv[pl.ds(g * _L, _L)] = inf16
        return c

    lax.fori_loop(0, _RS // _L, initg, 0)

    def cb_body(cb, c):
        c0 = cb * _CB
        bxs = [bxv[0, pl.ds(c0 + j * _L, _L)] for j in range(_CK)]
        bys = [byv[0, pl.ds(c0 + j * _L, _L)] for j in range(_CK)]
        bzs = [bzv[0, pl.ds(c0 + j * _L, _L)] for j in range(_CK)]

        def g_body(g, colaccs):
            ax16 = axv[0, pl.ds(g * _L, _L)]
            ay16 = ayv[0, pl.ds(g * _L, _L)]
            az16 = azv[0, pl.ds(g * _L, _L)]
            rvec = inf16
            accs = list(colaccs)
            for r in range(_L):
                axr = ax16[r]
                ayr = ay16[r]
                azr = az16[r]
                rr = None
                for j in range(_CK):
                    dx = bxs[j] - axr
                    dy = bys[j] - ayr
                    dz = bzs[j] - azr
                    d2 = dx * dx + dy * dy + dz * dz
                    accs[j] = jnp.minimum(accs[j], d2)
                    rr = d2 if rr is None else jnp.minimum(rr, d2)
                for idxv in xors:
                    rr = jnp.minimum(rr, _lane_perm(rr, idxv))
                rvec = jnp.where(lanes == r, rr, rvec)
            rminv[pl.ds(g * _L, _L)] = jnp.minimum(rminv[pl.ds(g * _L, _L)],
                                                   rvec)
            return tuple(accs)

        colaccs = lax.fori_loop(0, _RS // _L, g_body, (inf16,) * _CK)
        for j in range(_CK):
            cminv[pl.ds(c0 + j * _L, _L)] = colaccs[j]
        return c

    lax.fori_loop(0, _M // _CB, cb_body, 0)

    pltpu.sync_copy(rminv, row_o.at[box, pl.ds(r0, _RS)])
    pltpu.sync_copy(cminv, col_o.at[box, sh])


def _tp_body(a_ref, b_ref, at_ref, bt_ref):
    at_ref[0] = a_ref[0].T
    bt_ref[0] = b_ref[0].T


_tp = pl.pallas_call(
    _tp_body,
    grid=(_B,),
    in_specs=[
        pl.BlockSpec((1, _N, 3), lambda b: (b, 0, 0)),
        pl.BlockSpec((1, _M, 3), lambda b: (b, 0, 0)),
    ],
    out_specs=[
        pl.BlockSpec((1, 3, _N), lambda b: (b, 0, 0)),
        pl.BlockSpec((1, 3, _M), lambda b: (b, 0, 0)),
    ],
    out_shape=[
        jax.ShapeDtypeStruct((_B, 3, _N), jnp.float32),
        jax.ShapeDtypeStruct((_B, 3, _M), jnp.float32),
    ],
)


def _tc_min_body(a_ref, bt_ref, rowmin_ref, colminp_ref):
    axb = a_ref[0, :, 0:1]
    ayb = a_ref[0, :, 1:2]
    azb = a_ref[0, :, 2:3]
    bxb = bt_ref[0, 0:1, :]
    byb = bt_ref[0, 1:2, :]
    bzb = bt_ref[0, 2:3, :]
    d2 = (jnp.square(axb - bxb) + jnp.square(ayb - byb)
          + jnp.square(azb - bzb))
    rowmin_ref[0, 0, 0] = jnp.min(d2, axis=1)
    colminp_ref[0, 0, 0] = jnp.min(d2, axis=0)


_tc_min = pl.pallas_call(
    _tc_min_body,
    grid=(_B, _NRB),
    in_specs=[
        pl.BlockSpec((1, _RB, 3), lambda b, r: (b, (_NSC // _RB) + r, 0)),
        pl.BlockSpec((1, 3, _M), lambda b, r: (b, 0, 0)),
    ],
    out_specs=[
        pl.BlockSpec((1, 1, 1, _RB), lambda b, r: (b, r, 0, 0)),
        pl.BlockSpec((1, 1, 1, _M), lambda b, r: (b, r, 0, 0)),
    ],
    out_shape=[
        jax.ShapeDtypeStruct((_B, _NRB, 1, _RB), jnp.float32),
        jax.ShapeDtypeStruct((_B, _NRB, 1, _M), jnp.float32),
    ],
)


def _finish_body(rs_ref, rt_ref, cs_ref, ct_ref, out_ref):
    rt2 = rt_ref[...].reshape(_B, _NTC)
    rm2 = jnp.concatenate([rs_ref[...], rt2], axis=1)
    cm2 = jnp.minimum(jnp.min(cs_ref[...], axis=1),
                      jnp.min(ct_ref[...].reshape(_B, _NRB, _M), axis=1))
    rm = jnp.sqrt(jnp.maximum(rm2, 1e-12))
    cm = jnp.sqrt(jnp.maximum(cm2, 1e-12))
    lb = jnp.mean(cm, axis=1)
    lf = jnp.mean(rm, axis=1)
    lm = jnp.max(rm, axis=1)
    out_ref[...] = jnp.mean(5.0 * lb + lf + lm).reshape(1, 1)


_finish = pl.pallas_call(
    _finish_body,
    out_shape=jax.ShapeDtypeStruct((1, 1), jnp.float32),
)


@jax.jit
def kernel(sampled_lidar_list, surface_points):
    a = sampled_lidar_list
    b = surface_points
    at, bt = _tp(a, b)
    rowmin_sc, colmin_sc = _sc_min(at, bt)
    rowmin_tc, colmin_tc = _tc_min(a, bt)
    return _finish(rowmin_sc, rowmin_tc, colmin_sc, colmin_tc)[0, 0]
